# Initial kernel scaffold; baseline (speedup 1.0000x reference)
import jax
import jax.numpy as jnp
from jax.experimental import pallas as pl


def kernel(x_in, table):
    raise NotImplementedError("write your pallas kernel here")


# speedup vs baseline: 5.0029x; 5.0029x over previous
import functools

import numpy as np
import jax
import jax.numpy as jnp
from jax import lax
from jax.experimental import pallas as pl
from jax.experimental.pallas import tpu as pltpu
from jax.experimental.pallas import tpu_sc as plsc

_D = 16


def _pos_encoding(seq_len, embed_dim):
    pos = np.arange(seq_len)[:, np.newaxis]
    i = np.arange(embed_dim)[np.newaxis, :]
    angle_rates = 1.0 / np.power(10000, 2 * (i // 2) / np.float32(embed_dim))
    a = pos * angle_rates
    a[:, 0::2] = np.sin(a[:, 0::2])
    a[:, 1::2] = np.cos(a[:, 1::2])
    return jnp.asarray(a, dtype=jnp.float32)


@functools.lru_cache(maxsize=None)
def _build(n_rows, seq_len):
    nc, ns = 2, 16
    nw = nc * ns
    assert n_rows % nw == 0
    n_per_w = n_rows // nw
    seqs_per_chunk = 8
    ch = seqs_per_chunk * seq_len
    assert n_per_w % ch == 0
    n_chunks = n_per_w // ch

    mesh = plsc.VectorSubcoreMesh(core_axis_name="c", subcore_axis_name="s")

    @functools.partial(
        pl.kernel,
        out_type=jax.ShapeDtypeStruct((n_rows, _D), jnp.float32),
        mesh=mesh,
        scratch_types=[
            pltpu.VMEM((ch,), jnp.int32),
            pltpu.VMEM((ch, _D), jnp.float32),
            pltpu.VMEM((seq_len, _D), jnp.float32),
            pltpu.SemaphoreType.DMA,
        ],
        compiler_params=pltpu.CompilerParams(use_tc_tiling_on_sc=False),
    )
    def run(x_hbm, pos_hbm, table_hbm, out_hbm, idx_v, rows_v, pos_v, sem):
        c = lax.axis_index("c")
        s = lax.axis_index("s")
        wid = s * nc + c
        base = wid * n_per_w
        pltpu.sync_copy(pos_hbm, pos_v)

        def chunk_body(k, carry):
            off = base + k * ch
            pltpu.sync_copy(x_hbm.at[pl.ds(off, ch)], idx_v)
            pltpu.async_copy(table_hbm.at[idx_v], rows_v, sem).wait()

            def add_body(j, carry2):
                pv = pos_v[j]
                for t in range(seqs_per_chunk):
                    r = t * seq_len + j
                    rows_v[r] = rows_v[r] + pv
                return carry2

            lax.fori_loop(0, seq_len, add_body, 0)
            pltpu.sync_copy(rows_v, out_hbm.at[pl.ds(off, ch)])
            return carry

        lax.fori_loop(0, n_chunks, chunk_body, 0)

    return run


def kernel(x_in, table):
    b, s = x_in.shape
    n = b * s
    x_flat = x_in.reshape(n).astype(jnp.int32)
    pos = _pos_encoding(s, _D)
    out = _build(n, s)(x_flat, pos, table)
    return out.reshape(b, s, _D)


# speedup vs baseline: 5.3349x; 1.0664x over previous
import functools

import numpy as np
import jax
import jax.numpy as jnp
from jax import lax
from jax.experimental import pallas as pl
from jax.experimental.pallas import tpu as pltpu
from jax.experimental.pallas import tpu_sc as plsc

_D = 16


def _pos_encoding(seq_len, embed_dim):
    pos = np.arange(seq_len)[:, np.newaxis]
    i = np.arange(embed_dim)[np.newaxis, :]
    angle_rates = 1.0 / np.power(10000, 2 * (i // 2) / np.float32(embed_dim))
    a = pos * angle_rates
    a[:, 0::2] = np.sin(a[:, 0::2])
    a[:, 1::2] = np.cos(a[:, 1::2])
    return jnp.asarray(a, dtype=jnp.float32)


@functools.lru_cache(maxsize=None)
def _build(n_rows, seq_len):
    nc, ns = 2, 16
    nw = nc * ns
    assert n_rows % nw == 0
    n_per_w = n_rows // nw
    seqs_per_chunk = 8
    ch = seqs_per_chunk * seq_len
    assert n_per_w % ch == 0
    n_chunks = n_per_w // ch

    mesh = plsc.VectorSubcoreMesh(core_axis_name="c", subcore_axis_name="s")

    @functools.partial(
        pl.kernel,
        out_type=jax.ShapeDtypeStruct((n_rows, _D), jnp.float32),
        mesh=mesh,
        scratch_types=[
            pltpu.VMEM((n_per_w,), jnp.int32),
            pltpu.VMEM((ch, _D), jnp.float32),
            pltpu.VMEM((ch, _D), jnp.float32),
            pltpu.VMEM((seq_len, _D), jnp.float32),
            pltpu.SemaphoreType.DMA,
            pltpu.SemaphoreType.DMA,
            pltpu.SemaphoreType.DMA,
            pltpu.SemaphoreType.DMA,
        ],
        compiler_params=pltpu.CompilerParams(use_tc_tiling_on_sc=False),
    )
    def run(x_hbm, pos_hbm, table_hbm, out_hbm, idx_v, rows0, rows1, pos_v,
            gsem0, gsem1, osem0, osem1):
        c = lax.axis_index("c")
        s = lax.axis_index("s")
        wid = s * nc + c
        base = wid * n_per_w
        pltpu.sync_copy(pos_hbm, pos_v)
        pltpu.sync_copy(x_hbm.at[pl.ds(base, n_per_w)], idx_v)

        bufs = (rows0, rows1)
        gsems = (gsem0, gsem1)
        osems = (osem0, osem1)
        gathers = [None] * n_chunks
        outs = [None] * n_chunks

        def start_gather(k):
            b = k % 2
            gathers[k] = pltpu.async_copy(
                table_hbm.at[idx_v.at[pl.ds(k * ch, ch)]], bufs[b], gsems[b])

        start_gather(0)
        for k in range(n_chunks):
            b = k % 2
            buf = bufs[b]
            if k + 1 < n_chunks:
                if k >= 1:
                    outs[k - 1].wait()
                start_gather(k + 1)
            gathers[k].wait()

            @plsc.parallel_loop(0, seq_len, unroll=4)
            def add_body(j):
                pv = pos_v[j]
                for t in range(seqs_per_chunk):
                    r = t * seq_len + j
                    buf[r] = buf[r] + pv

            outs[k] = pltpu.async_copy(
                buf, out_hbm.at[pl.ds(base + k * ch, ch)], osems[b])
        outs[n_chunks - 2].wait()
        outs[n_chunks - 1].wait()

    return run


def kernel(x_in, table):
    b, s = x_in.shape
    n = b * s
    x_flat = x_in.reshape(n).astype(jnp.int32)
    pos = _pos_encoding(s, _D)
    out = _build(n, s)(x_flat, pos, table)
    return out.reshape(b, s, _D)


# speedup vs baseline: 9.2067x; 1.7258x over previous
import functools

import numpy as np
import jax
import jax.numpy as jnp
from jax import lax
from jax.experimental import pallas as pl
from jax.experimental.pallas import tpu as pltpu
from jax.experimental.pallas import tpu_sc as plsc

_D = 16
_LANES = 128


def _pos_encoding(seq_len, embed_dim):
    pos = np.arange(seq_len)[:, np.newaxis]
    i = np.arange(embed_dim)[np.newaxis, :]
    angle_rates = 1.0 / np.power(10000, 2 * (i // 2) / np.float32(embed_dim))
    a = pos * angle_rates
    a[:, 0::2] = np.sin(a[:, 0::2])
    a[:, 1::2] = np.cos(a[:, 1::2])
    return a.astype(np.float32)


@functools.lru_cache(maxsize=None)
def _build(seq_len, batch, vocab):
    nc, ns = 2, 16
    nw = nc * ns
    assert batch == nw * _LANES and seq_len % 2 == 0
    n_dg = _D // 8

    mesh = plsc.VectorSubcoreMesh(core_axis_name="c", subcore_axis_name="s")

    @functools.partial(
        pl.kernel,
        out_type=jax.ShapeDtypeStruct((seq_len, n_dg, nw, 8, _LANES),
                                      jnp.float32),
        mesh=mesh,
        scratch_types=[
            pltpu.VMEM((2, _LANES), jnp.int32),
            pltpu.VMEM((2, _LANES, _D), jnp.float32),
            pltpu.VMEM((2, n_dg, 8, _LANES), jnp.float32),
            pltpu.VMEM((seq_len * _D, _D), jnp.float32),
            pltpu.SemaphoreType.DMA,
            pltpu.SemaphoreType.DMA,
            pltpu.SemaphoreType.DMA,
            pltpu.SemaphoreType.DMA,
            pltpu.SemaphoreType.DMA,
        ],
        compiler_params=pltpu.CompilerParams(
            use_tc_tiling_on_sc=False, needs_layout_passes=False),
    )
    def run(x_hbm, psp_hbm, table_hbm, out_hbm, ibuf, gbuf, obuf, psp_v,
            isem0, isem1, gsem, osem0, osem1):
        w = lax.axis_index("s") * nc + lax.axis_index("c")
        col0 = w * _LANES
        isems = (isem0, isem1)
        osems = (osem0, osem1)
        pltpu.sync_copy(psp_hbm, psp_v)

        def idx_copy(s, b):
            return pltpu.async_copy(
                x_hbm.at[s].at[pl.ds(col0, _LANES)], ibuf.at[b], isems[b])

        def gather_copy(s, b):
            del s
            return pltpu.async_copy(
                table_hbm.at[ibuf.at[b]], gbuf.at[b], gsem)

        def out_copy(s, b, dg):
            return pltpu.async_copy(
                obuf.at[b].at[dg], out_hbm.at[s, dg, w], osems[b])

        idx_copy(0, 0)
        idx_copy(1, 1)
        pltpu.make_async_copy(
            x_hbm.at[0].at[pl.ds(col0, _LANES)], ibuf.at[0], isems[0]).wait()
        gather_copy(0, 0)

        rowj = []
        for j in range(8):
            rowj.append(lax.iota(jnp.int32, 16) + 16 * j)

        def compute(s, b):
            g = gbuf.at[b]
            for dg in range(n_dg):
                for r in range(8):
                    d = dg * 8 + r
                    pv = psp_v[s * _D + d]
                    cold = jnp.full((16,), d, jnp.int32)
                    for j in range(8):
                        vals = plsc.load_gather(g, [rowj[j], cold])
                        obuf[b, dg, r, pl.ds(j * 16, 16)] = vals + pv

        def phase(s, b):
            pltpu.make_async_copy(
                table_hbm.at[ibuf.at[b]], gbuf.at[b], gsem).wait()
            @pl.when(s + 2 < seq_len)
            def _():
                idx_copy(s + 2, b)
            @pl.when(s + 1 < seq_len)
            def _():
                pltpu.make_async_copy(
                    x_hbm.at[s + 1].at[pl.ds(col0, _LANES)], ibuf.at[1 - b],
                    isems[1 - b]).wait()
                gather_copy(s + 1, 1 - b)
            @pl.when(s >= 2)
            def _():
                for dg in range(n_dg):
                    pltpu.make_async_copy(
                        obuf.at[b].at[dg], out_hbm.at[s - 2, dg, w],
                        osems[b]).wait()
            compute(s, b)
            for dg in range(n_dg):
                out_copy(s, b, dg)

        def pair_body(s2, carry):
            s = 2 * s2
            phase(s, 0)
            phase(s + 1, 1)
            return carry

        lax.fori_loop(0, seq_len // 2, pair_body, 0)
        for s, b in ((seq_len - 2, 0), (seq_len - 1, 1)):
            for dg in range(n_dg):
                pltpu.make_async_copy(
                    obuf.at[b].at[dg], out_hbm.at[s, dg, w], osems[b]).wait()

    return run


def kernel(x_in, table):
    b, s = x_in.shape
    vocab, d = table.shape
    x_t = x_in.T.astype(jnp.int32)
    pos = _pos_encoding(s, d)
    psp = jnp.asarray(np.repeat(pos.reshape(-1), 16).reshape(-1, 16))
    out_lin = _build(s, b, vocab)(x_t, psp, table)
    nw = out_lin.shape[2]
    return out_lin.transpose((2, 4, 0, 1, 3)).reshape(nw * _LANES, s, d)


# speedup vs baseline: 11.6218x; 1.2623x over previous
import functools

import numpy as np
import jax
import jax.numpy as jnp
from jax import lax
from jax.experimental import pallas as pl
from jax.experimental.pallas import tpu as pltpu
from jax.experimental.pallas import tpu_sc as plsc

_D = 16
_LANES = 128


def _pos_encoding(seq_len, embed_dim):
    pos = np.arange(seq_len)[:, np.newaxis]
    i = np.arange(embed_dim)[np.newaxis, :]
    angle_rates = 1.0 / np.power(10000, 2 * (i // 2) / np.float32(embed_dim))
    a = pos * angle_rates
    a[:, 0::2] = np.sin(a[:, 0::2])
    a[:, 1::2] = np.cos(a[:, 1::2])
    return a.astype(np.float32)


@functools.lru_cache(maxsize=None)
def _build(seq_len, batch, vocab):
    nc, ns = 2, 16
    nw = nc * ns
    assert batch == nw * _LANES and seq_len % 2 == 0
    n_dg = _D // 8

    mesh = plsc.VectorSubcoreMesh(core_axis_name="c", subcore_axis_name="s")

    @functools.partial(
        pl.kernel,
        out_type=jax.ShapeDtypeStruct((seq_len, n_dg, nw, 8, _LANES),
                                      jnp.float32),
        mesh=mesh,
        scratch_types=[
            pltpu.VMEM((2, _LANES), jnp.int32),
            pltpu.VMEM((2, _LANES, _D), jnp.float32),
            pltpu.VMEM((2, n_dg, 8, _LANES), jnp.float32),
            pltpu.VMEM((seq_len * _D, _D), jnp.float32),
            pltpu.SemaphoreType.DMA,
            pltpu.SemaphoreType.DMA,
            pltpu.SemaphoreType.DMA,
            pltpu.SemaphoreType.DMA,
            pltpu.SemaphoreType.DMA,
        ],
        compiler_params=pltpu.CompilerParams(
            use_tc_tiling_on_sc=False, needs_layout_passes=False),
    )
    def run(x_hbm, psp_hbm, table_hbm, out_hbm, ibuf, gbuf, obuf, psp_v,
            isem0, isem1, gsem, osem0, osem1):
        w = lax.axis_index("s") * nc + lax.axis_index("c")
        col0 = w * _LANES
        isems = (isem0, isem1)
        osems = (osem0, osem1)
        pltpu.sync_copy(psp_hbm, psp_v)

        def idx_copy(s, b):
            return pltpu.async_copy(
                x_hbm.at[s].at[pl.ds(col0, _LANES)], ibuf.at[b], isems[b])

        def gather_copy(s, b):
            del s
            return pltpu.async_copy(
                table_hbm.at[ibuf.at[b]], gbuf.at[b], gsem)

        def out_copy(s, b, dg):
            return pltpu.async_copy(
                obuf.at[b].at[dg], out_hbm.at[s, dg, w], osems[b])

        idx_copy(0, 0)
        idx_copy(1, 1)
        pltpu.make_async_copy(
            x_hbm.at[0].at[pl.ds(col0, _LANES)], ibuf.at[0], isems[0]).wait()
        gather_copy(0, 0)

        iota16 = lax.iota(jnp.int32, 16)

        def compute(s, b):
            g = gbuf.at[b]

            @plsc.parallel_loop(0, _D * 8, unroll=4)
            def vloop(k):
                dg = k // 64
                r = (k // 8) % 8
                j = k % 8
                d = dg * 8 + r
                rows = iota16 + j * 16
                cold = jnp.full((16,), d, jnp.int32)
                vals = plsc.load_gather(g, [rows, cold])
                pv = psp_v[s * _D + d]
                obuf[b, dg, r, pl.ds(j * 16, 16)] = vals + pv

        def phase(s, b):
            pltpu.make_async_copy(
                table_hbm.at[ibuf.at[b]], gbuf.at[b], gsem).wait()
            @pl.when(s + 2 < seq_len)
            def _():
                idx_copy(s + 2, b)
            @pl.when(s + 1 < seq_len)
            def _():
                pltpu.make_async_copy(
                    x_hbm.at[s + 1].at[pl.ds(col0, _LANES)], ibuf.at[1 - b],
                    isems[1 - b]).wait()
                gather_copy(s + 1, 1 - b)
            @pl.when(s >= 2)
            def _():
                for dg in range(n_dg):
                    pltpu.make_async_copy(
                        obuf.at[b].at[dg], out_hbm.at[s - 2, dg, w],
                        osems[b]).wait()
            compute(s, b)
            for dg in range(n_dg):
                out_copy(s, b, dg)

        def pair_body(s2, carry):
            s = 2 * s2
            phase(s, 0)
            phase(s + 1, 1)
            return carry

        lax.fori_loop(0, seq_len // 2, pair_body, 0)
        for s, b in ((seq_len - 2, 0), (seq_len - 1, 1)):
            for dg in range(n_dg):
                pltpu.make_async_copy(
                    obuf.at[b].at[dg], out_hbm.at[s, dg, w], osems[b]).wait()

    return run


def kernel(x_in, table):
    b, s = x_in.shape
    vocab, d = table.shape
    x_t = x_in.T.astype(jnp.int32)
    pos = _pos_encoding(s, d)
    psp = jnp.asarray(np.repeat(pos.reshape(-1), 16).reshape(-1, 16))
    out_lin = _build(s, b, vocab)(x_t, psp, table)
    nw = out_lin.shape[2]
    return out_lin.transpose((2, 4, 0, 1, 3)).reshape(nw * _LANES, s, d)


# speedup vs baseline: 12.0511x; 1.0369x over previous
import functools

import numpy as np
import jax
import jax.numpy as jnp
from jax import lax
from jax.experimental import pallas as pl
from jax.experimental.pallas import tpu as pltpu
from jax.experimental.pallas import tpu_sc as plsc

_D = 16
_LANES = 128


def _pos_encoding(seq_len, embed_dim):
    pos = np.arange(seq_len)[:, np.newaxis]
    i = np.arange(embed_dim)[np.newaxis, :]
    angle_rates = 1.0 / np.power(10000, 2 * (i // 2) / np.float32(embed_dim))
    a = pos * angle_rates
    a[:, 0::2] = np.sin(a[:, 0::2])
    a[:, 1::2] = np.cos(a[:, 1::2])
    return a.astype(np.float32)


@functools.lru_cache(maxsize=None)
def _build(seq_len, batch, vocab):
    nc, ns = 2, 16
    nw = nc * ns
    assert batch == nw * _LANES and seq_len % 2 == 0
    n_dg = _D // 8

    mesh = plsc.VectorSubcoreMesh(core_axis_name="c", subcore_axis_name="s")

    @functools.partial(
        pl.kernel,
        out_type=jax.ShapeDtypeStruct((seq_len, n_dg, nw, 8 * _LANES),
                                      jnp.float32),
        mesh=mesh,
        scratch_types=[
            pltpu.VMEM((2, _LANES), jnp.int32),
            pltpu.VMEM((2, _LANES, _D), jnp.float32),
            pltpu.VMEM((2, n_dg * 8 * _LANES), jnp.float32),
            pltpu.VMEM((seq_len, _D), jnp.float32),
            pltpu.SemaphoreType.DMA,
            pltpu.SemaphoreType.DMA,
            pltpu.SemaphoreType.DMA,
            pltpu.SemaphoreType.DMA,
            pltpu.SemaphoreType.DMA,
        ],
        compiler_params=pltpu.CompilerParams(
            use_tc_tiling_on_sc=False, needs_layout_passes=False),
    )
    def run(x_hbm, pos_hbm, table_hbm, out_hbm, ibuf, gbuf, obuf, pos_v,
            isem0, isem1, gsem, osem0, osem1):
        w = lax.axis_index("s") * nc + lax.axis_index("c")
        col0 = w * _LANES
        isems = (isem0, isem1)
        osems = (osem0, osem1)
        pltpu.sync_copy(pos_hbm, pos_v)

        def idx_copy(s, b):
            return pltpu.async_copy(
                x_hbm.at[s].at[pl.ds(col0, _LANES)], ibuf.at[b], isems[b])

        def gather_copy(s, b):
            del s
            return pltpu.async_copy(
                table_hbm.at[ibuf.at[b]], gbuf.at[b], gsem)

        def out_copy(s, b, dg):
            return pltpu.async_copy(
                obuf.at[b].at[pl.ds(dg * 8 * _LANES, 8 * _LANES)],
                out_hbm.at[s, dg, w], osems[b])

        idx_copy(0, 0)
        idx_copy(1, 1)
        pltpu.make_async_copy(
            x_hbm.at[0].at[pl.ds(col0, _LANES)], ibuf.at[0], isems[0]).wait()
        gather_copy(0, 0)

        iota16 = lax.iota(jnp.int32, 16)
        sidx = (iota16 // 8) * (8 * _LANES) + (iota16 % 8) * _LANES

        def compute(s, b):
            g = gbuf.at[b]
            ofl = obuf.at[b]
            pv = pos_v[s]

            @plsc.parallel_loop(0, _LANES, unroll=8)
            def vloop(i):
                vals = g[i] + pv
                plsc.store_scatter(ofl, [sidx + i], vals)

        def phase(s, b):
            pltpu.make_async_copy(
                table_hbm.at[ibuf.at[b]], gbuf.at[b], gsem).wait()
            @pl.when(s + 2 < seq_len)
            def _():
                idx_copy(s + 2, b)
            @pl.when(s + 1 < seq_len)
            def _():
                pltpu.make_async_copy(
                    x_hbm.at[s + 1].at[pl.ds(col0, _LANES)], ibuf.at[1 - b],
                    isems[1 - b]).wait()
                gather_copy(s + 1, 1 - b)
            @pl.when(s >= 2)
            def _():
                for dg in range(n_dg):
                    pltpu.make_async_copy(
                        obuf.at[b].at[pl.ds(dg * 8 * _LANES, 8 * _LANES)],
                        out_hbm.at[s - 2, dg, w], osems[b]).wait()
            compute(s, b)
            for dg in range(n_dg):
                out_copy(s, b, dg)

        def pair_body(s2, carry):
            s = 2 * s2
            phase(s, 0)
            phase(s + 1, 1)
            return carry

        lax.fori_loop(0, seq_len // 2, pair_body, 0)
        for s, b in ((seq_len - 2, 0), (seq_len - 1, 1)):
            for dg in range(n_dg):
                pltpu.make_async_copy(
                    obuf.at[b].at[pl.ds(dg * 8 * _LANES, 8 * _LANES)],
                    out_hbm.at[s, dg, w], osems[b]).wait()

    return run


def kernel(x_in, table):
    b, s = x_in.shape
    vocab, d = table.shape
    x_t = x_in.T.astype(jnp.int32)
    pos = jnp.asarray(_pos_encoding(s, d))
    out_lin = _build(s, b, vocab)(x_t, pos, table)
    nw = out_lin.shape[2]
    out5 = out_lin.reshape(s, d // 8, nw, 8, _LANES)
    return out5.transpose((2, 4, 0, 1, 3)).reshape(nw * _LANES, s, d)
